# baseline (speedup 1.0000x reference)
import functools

import jax
import jax.numpy as jnp
from jax import lax
from jax.experimental import pallas as pl
from jax.experimental.pallas import tpu as pltpu
from jax.experimental.pallas import tpu_sc as plsc

DIM = 64
BATCH = 16384


@functools.lru_cache(maxsize=None)
def _build():
    info = plsc.get_sparse_core_info()
    nc, ns = info.num_cores, info.num_subcores
    nw = nc * ns
    bpw = BATCH // nw

    mesh = plsc.VectorSubcoreMesh(core_axis_name="c", subcore_axis_name="s")
    out_row = jax.ShapeDtypeStruct((BATCH, DIM), jnp.float32)

    @functools.partial(
        pl.kernel,
        mesh=mesh,
        out_type=(out_row, out_row, out_row),
        compiler_params=pltpu.CompilerParams(use_tc_tiling_on_sc=False),
        scratch_types=[
            pltpu.VMEM((bpw,), jnp.int32),
            pltpu.VMEM((bpw,), jnp.int32),
            pltpu.VMEM((bpw,), jnp.int32),
            pltpu.VMEM((bpw, DIM), jnp.float32),
            pltpu.VMEM((bpw, DIM), jnp.float32),
            pltpu.VMEM((bpw, DIM), jnp.float32),
            pltpu.SemaphoreType.DMA,
        ],
    )
    def k(h_hbm, r_hbm, t_hbm, ent_hbm, rel_hbm, out_h, out_r, out_t,
          idx_h, idx_r, idx_t, rows_h, rows_r, rows_t, sem):
        wid = lax.axis_index("s") * nc + lax.axis_index("c")
        base = wid * bpw
        pltpu.sync_copy(h_hbm.at[pl.ds(base, bpw)], idx_h)
        pltpu.sync_copy(r_hbm.at[pl.ds(base, bpw)], idx_r)
        pltpu.sync_copy(t_hbm.at[pl.ds(base, bpw)], idx_t)
        ch = pltpu.async_copy(ent_hbm.at[idx_h], rows_h, sem)
        cr = pltpu.async_copy(rel_hbm.at[idx_r], rows_r, sem)
        ct = pltpu.async_copy(ent_hbm.at[idx_t], rows_t, sem)
        ch.wait()
        cr.wait()
        ct.wait()
        pltpu.sync_copy(rows_h, out_h.at[pl.ds(base, bpw)])
        pltpu.sync_copy(rows_r, out_r.at[pl.ds(base, bpw)])
        pltpu.sync_copy(rows_t, out_t.at[pl.ds(base, bpw)])

    return k


def kernel(x, ent_emb, rel_emb):
    xi = jnp.asarray(x, jnp.int32)
    head, rel, tail = xi[:, 0], xi[:, 1], xi[:, 2]
    return _build()(head, rel, tail, ent_emb, rel_emb)
